# Initial kernel scaffold; baseline (speedup 1.0000x reference)
import jax
import jax.numpy as jnp
from jax.experimental import pallas as pl


def kernel(x, edge_index, edge_weight):
    raise NotImplementedError("write your pallas kernel here")


# speedup vs baseline: 3.7603x; 3.7603x over previous
import functools

import jax
import jax.numpy as jnp
from jax import lax
from jax.experimental import pallas as pl
from jax.experimental.pallas import tpu as pltpu
from jax.experimental.pallas import tpu_sc as plsc

N = 10000
D = 128
E = 320000
NC = 2
NS = 16
NW = NC * NS
C = 128
EPW = 10240
NCHUNK = EPW // C
E2 = NW * EPW
NP = 10240
ZROW = NP // NS
L = 16


def _hop_body(with_deg, table, ed, *rest):
    if with_deg:
        out, degout, ebuf, rows, degl, acc, sem = rest
    else:
        out, ebuf, rows, acc, sem = rest
    cid = lax.axis_index("c")
    sid = lax.axis_index("s")
    wid = sid * NC + cid
    zv = jnp.zeros((L,), jnp.float32)

    def zero_rows(i, c):
        for f in range(D // L):
            rows[i, pl.ds(f * L, L)] = zv
        return c

    lax.fori_loop(0, C, zero_rows, 0)
    for k in range(ZROW // C):
        pltpu.sync_copy(rows, acc.at[pl.ds(sid * ZROW + k * C, C)])
    if with_deg:
        def zero_deg(i, c):
            degl[pl.ds(i * L, L)] = zv
            return c

        lax.fori_loop(0, NP // L, zero_deg, 0)
    plsc.subcore_barrier()

    def chunk(j, c):
        pltpu.sync_copy(ed.at[wid, j], ebuf)
        pltpu.async_copy(table.at[ebuf.at[0]], rows, sem).wait()

        def scale(g, c2):
            wv16 = plsc.bitcast(ebuf[2, pl.ds(g * L, L)], jnp.float32)
            for i in range(L):
                wv = jnp.full((L,), wv16[i], jnp.float32)
                e = g * L + i
                for f in range(D // L):
                    rows[e, pl.ds(f * L, L)] = rows[e, pl.ds(f * L, L)] * wv
            if with_deg:
                dv = ebuf[1, pl.ds(g * L, L)]
                plsc.addupdate_scatter(degl, [dv], wv16)
            return c2

        lax.fori_loop(0, C // L, scale, 0)
        pltpu.sync_copy(rows, acc.at[ebuf.at[1]], add=True)
        return c

    lax.fori_loop(0, NCHUNK, chunk, 0)
    plsc.subcore_barrier()

    for k in range(ZROW // C):
        sl = pl.ds(sid * ZROW + k * C, C)
        pltpu.sync_copy(acc.at[sl], out.at[cid, sl])
    if with_deg:
        pltpu.sync_copy(degl, degout.at[wid])


def _make_hop(with_deg):
    mesh = plsc.VectorSubcoreMesh(core_axis_name="c", subcore_axis_name="s")
    out_type = [jax.ShapeDtypeStruct((NC, NP, D), jnp.float32)]
    if with_deg:
        out_type.append(jax.ShapeDtypeStruct((NW, NP), jnp.float32))
    scratch = [
        pltpu.VMEM((3, C), jnp.int32),
        pltpu.VMEM((C, D), jnp.float32),
    ]
    if with_deg:
        scratch.append(pltpu.VMEM((NP,), jnp.float32))
    scratch.append(pltpu.VMEM_SHARED((NP, D), jnp.float32))
    scratch.append(pltpu.SemaphoreType.DMA)
    return pl.kernel(
        functools.partial(_hop_body, with_deg),
        out_type=out_type,
        mesh=mesh,
        scratch_types=scratch,
        compiler_params=pltpu.CompilerParams(needs_layout_passes=False),
    )


_hop_with_deg = _make_hop(True)
_hop_no_deg = _make_hop(False)

BR = 512
_ONES = None


def _comb1_body(p0, p1, dg, xb, x1o, dio, go):
    ones = jnp.ones((NW, 1), jnp.float32)
    deg = lax.dot_general(dg[...], ones, (((0,), (0,)), ((), ())),
                          preferred_element_type=jnp.float32)
    dinv = jnp.where(deg > 0, 1.0 / deg, 0.0)
    x1o[...] = (p0[...] + p1[...]) * dinv
    dio[...] = dinv
    i = pl.program_id(0)

    @pl.when(i == 0)
    def _():
        go[...] = jnp.zeros_like(go)

    go[...] += jnp.sum(xb[...], axis=0, keepdims=True)

    @pl.when(i == pl.num_programs(0) - 1)
    def _():
        go[...] = go[...] * (1.0 / N)


_comb1 = pl.pallas_call(
    _comb1_body,
    grid=(NP // BR,),
    in_specs=[
        pl.BlockSpec((BR, D), lambda i: (i, 0)),
        pl.BlockSpec((BR, D), lambda i: (i, 0)),
        pl.BlockSpec((NW, BR), lambda i: (0, i)),
        pl.BlockSpec((BR, D), lambda i: (i, 0)),
    ],
    out_specs=[
        pl.BlockSpec((BR, D), lambda i: (i, 0)),
        pl.BlockSpec((BR, 1), lambda i: (i, 0)),
        pl.BlockSpec((1, D), lambda i: (0, 0)),
    ],
    out_shape=[
        jax.ShapeDtypeStruct((NP, D), jnp.float32),
        jax.ShapeDtypeStruct((NP, 1), jnp.float32),
        jax.ShapeDtypeStruct((1, D), jnp.float32),
    ],
)


def _comb2_body(p0, p1, di, x2o):
    x2o[...] = (p0[...] + p1[...]) * di[...]


_comb2 = pl.pallas_call(
    _comb2_body,
    grid=(NP // BR,),
    in_specs=[
        pl.BlockSpec((BR, D), lambda i: (i, 0)),
        pl.BlockSpec((BR, D), lambda i: (i, 0)),
        pl.BlockSpec((BR, 1), lambda i: (i, 0)),
    ],
    out_specs=pl.BlockSpec((BR, D), lambda i: (i, 0)),
    out_shape=jax.ShapeDtypeStruct((NP, D), jnp.float32),
)


def kernel(x, edge_index, edge_weight):
    src = edge_index[0]
    dst = edge_index[1]
    pad = E2 - E
    src3 = jnp.concatenate([src, jnp.zeros((pad,), jnp.int32)]).reshape(NW, NCHUNK, C)
    dst3 = jnp.concatenate([dst, jnp.zeros((pad,), jnp.int32)]).reshape(NW, NCHUNK, C)
    ewb = jax.lax.bitcast_convert_type(
        jnp.concatenate([edge_weight, jnp.zeros((pad,), jnp.float32)]),
        jnp.int32).reshape(NW, NCHUNK, C)
    ed = jnp.stack([src3, dst3, ewb], axis=2)
    xpad = jnp.concatenate([x, jnp.zeros((NP - N, D), jnp.float32)], axis=0)

    p1, degp = _hop_with_deg(xpad, ed)
    x1, dinv, g = _comb1(p1[0], p1[1], degp, xpad)
    (p2,) = _hop_no_deg(x1, ed)
    x2 = _comb2(p2[0], p2[1], dinv)

    return jnp.concatenate(
        [x, x1[:N], x2[:N], jnp.broadcast_to(g, (N, D))], axis=-1)


# speedup vs baseline: 4.5989x; 1.2230x over previous
import functools

import jax
import jax.numpy as jnp
from jax import lax
from jax.experimental import pallas as pl
from jax.experimental.pallas import tpu as pltpu
from jax.experimental.pallas import tpu_sc as plsc

N = 10000
D = 128
E = 320000
NC = 2
NS = 16
NW = NC * NS
C = 128
EPW = 10240
NCHUNK = EPW // C
E2 = NW * EPW
NP = 10240
ZROW = NP // NS
L = 16


def _hop_body(with_deg, table, ed, *rest):
    if with_deg:
        out, degout, ebuf0, ebuf1, dbuf0, dbuf1, rows0, rows1, degl, acc, \
            semE0, semE1, semG0, semG1, semS0, semS1 = rest
    else:
        out, ebuf0, ebuf1, dbuf0, dbuf1, rows0, rows1, acc, \
            semE0, semE1, semG0, semG1, semS0, semS1 = rest
    ebuf = (ebuf0, ebuf1)
    dbuf = (dbuf0, dbuf1)
    rows = (rows0, rows1)
    semE = (semE0, semE1)
    semG = (semG0, semG1)
    semS = (semS0, semS1)
    cid = lax.axis_index("c")
    sid = lax.axis_index("s")
    wid = sid * NC + cid
    zv = jnp.zeros((L,), jnp.float32)

    def zero_rows(i, c):
        for f in range(D // L):
            rows0[i, pl.ds(f * L, L)] = zv
        return c

    lax.fori_loop(0, C, zero_rows, 0)
    for k in range(ZROW // C):
        pltpu.sync_copy(rows0, acc.at[pl.ds(sid * ZROW + k * C, C)])
    if with_deg:
        def zero_deg(i, c):
            degl[pl.ds(i * L, L)] = zv
            return c

        lax.fori_loop(0, NP // L, zero_deg, 0)
    plsc.subcore_barrier()

    def _wait_e(b):
        pltpu.make_async_copy(ed.at[wid, 0], ebuf[b], semE[b]).wait()

    def _wait_g(b):
        pltpu.make_async_copy(table.at[pl.ds(0, C)], rows[b], semG[b]).wait()

    def _wait_s(b):
        pltpu.make_async_copy(rows[b], out.at[cid, pl.ds(0, C)],
                              semS[b]).wait()

    pltpu.async_copy(ed.at[wid, 0], ebuf[0], semE[0])
    pltpu.async_copy(ed.at[wid, 1], ebuf[1], semE[1])
    _wait_e(0)
    pltpu.async_copy(table.at[ebuf[0].at[0]], rows[0], semG[0])

    def chunk(j2, c):
        for b in range(2):
            j = j2 * 2 + b
            nb = 1 - b
            _wait_g(b)

            @pl.when((j >= 1) & (j + 1 < NCHUNK))
            def _():
                _wait_s(nb)

            @pl.when(j + 1 < NCHUNK)
            def _():
                _wait_e(nb)
                pltpu.async_copy(table.at[ebuf[nb].at[0]], rows[nb],
                                 semG[nb])

            def scale(g, c2):
                wv16 = plsc.bitcast(ebuf[b][2, pl.ds(g * L, L)], jnp.float32)
                for i in range(L):
                    wv = jnp.full((L,), wv16[i], jnp.float32)
                    e = g * L + i
                    for f in range(D // L):
                        rows[b][e, pl.ds(f * L, L)] = (
                            rows[b][e, pl.ds(f * L, L)] * wv)
                dv = ebuf[b][1, pl.ds(g * L, L)]
                dbuf[b][pl.ds(g * L, L)] = dv
                if with_deg:
                    plsc.addupdate_scatter(degl, [dv], wv16)
                return c2

            lax.fori_loop(0, C // L, scale, 0)
            pltpu.async_copy(rows[b], acc.at[dbuf[b]], semS[b],
                             add=True)

            @pl.when(j + 2 < NCHUNK)
            def _():
                pltpu.async_copy(ed.at[wid, j + 2], ebuf[b], semE[b])
        return c

    lax.fori_loop(0, NCHUNK // 2, chunk, 0)
    _wait_s(0)
    _wait_s(1)
    plsc.subcore_barrier()

    for k in range(ZROW // C):
        sl = pl.ds(sid * ZROW + k * C, C)
        pltpu.sync_copy(acc.at[sl], out.at[cid, sl])
    if with_deg:
        pltpu.sync_copy(degl, degout.at[wid])


def _make_hop(with_deg):
    mesh = plsc.VectorSubcoreMesh(core_axis_name="c", subcore_axis_name="s")
    out_type = [jax.ShapeDtypeStruct((NC, NP, D), jnp.float32)]
    if with_deg:
        out_type.append(jax.ShapeDtypeStruct((NW, NP), jnp.float32))
    scratch = [
        pltpu.VMEM((3, C), jnp.int32),
        pltpu.VMEM((3, C), jnp.int32),
        pltpu.VMEM((C,), jnp.int32),
        pltpu.VMEM((C,), jnp.int32),
        pltpu.VMEM((C, D), jnp.float32),
        pltpu.VMEM((C, D), jnp.float32),
    ]
    if with_deg:
        scratch.append(pltpu.VMEM((NP,), jnp.float32))
    scratch.append(pltpu.VMEM_SHARED((NP, D), jnp.float32))
    scratch.extend([pltpu.SemaphoreType.DMA] * 6)
    return pl.kernel(
        functools.partial(_hop_body, with_deg),
        out_type=out_type,
        mesh=mesh,
        scratch_types=scratch,
        compiler_params=pltpu.CompilerParams(needs_layout_passes=False),
    )


_hop_with_deg = _make_hop(True)
_hop_no_deg = _make_hop(False)

BR = 512
_ONES = None


def _comb1_body(p0, p1, dg, xb, x1o, dio, go):
    ones = jnp.ones((NW, 1), jnp.float32)
    deg = lax.dot_general(dg[...], ones, (((0,), (0,)), ((), ())),
                          preferred_element_type=jnp.float32)
    dinv = jnp.where(deg > 0, 1.0 / deg, 0.0)
    x1o[...] = (p0[...] + p1[...]) * dinv
    dio[...] = dinv
    i = pl.program_id(0)

    @pl.when(i == 0)
    def _():
        go[...] = jnp.zeros_like(go)

    go[...] += jnp.sum(xb[...], axis=0, keepdims=True)

    @pl.when(i == pl.num_programs(0) - 1)
    def _():
        go[...] = go[...] * (1.0 / N)


_comb1 = pl.pallas_call(
    _comb1_body,
    grid=(NP // BR,),
    in_specs=[
        pl.BlockSpec((BR, D), lambda i: (i, 0)),
        pl.BlockSpec((BR, D), lambda i: (i, 0)),
        pl.BlockSpec((NW, BR), lambda i: (0, i)),
        pl.BlockSpec((BR, D), lambda i: (i, 0)),
    ],
    out_specs=[
        pl.BlockSpec((BR, D), lambda i: (i, 0)),
        pl.BlockSpec((BR, 1), lambda i: (i, 0)),
        pl.BlockSpec((1, D), lambda i: (0, 0)),
    ],
    out_shape=[
        jax.ShapeDtypeStruct((NP, D), jnp.float32),
        jax.ShapeDtypeStruct((NP, 1), jnp.float32),
        jax.ShapeDtypeStruct((1, D), jnp.float32),
    ],
)


def _comb2_body(p0, p1, di, x2o):
    x2o[...] = (p0[...] + p1[...]) * di[...]


_comb2 = pl.pallas_call(
    _comb2_body,
    grid=(NP // BR,),
    in_specs=[
        pl.BlockSpec((BR, D), lambda i: (i, 0)),
        pl.BlockSpec((BR, D), lambda i: (i, 0)),
        pl.BlockSpec((BR, 1), lambda i: (i, 0)),
    ],
    out_specs=pl.BlockSpec((BR, D), lambda i: (i, 0)),
    out_shape=jax.ShapeDtypeStruct((NP, D), jnp.float32),
)


def kernel(x, edge_index, edge_weight):
    src = edge_index[0]
    dst = edge_index[1]
    pad = E2 - E
    src3 = jnp.concatenate([src, jnp.zeros((pad,), jnp.int32)]).reshape(NW, NCHUNK, C)
    dst3 = jnp.concatenate([dst, jnp.zeros((pad,), jnp.int32)]).reshape(NW, NCHUNK, C)
    ewb = jax.lax.bitcast_convert_type(
        jnp.concatenate([edge_weight, jnp.zeros((pad,), jnp.float32)]),
        jnp.int32).reshape(NW, NCHUNK, C)
    ed = jnp.stack([src3, dst3, ewb], axis=2)
    xpad = jnp.concatenate([x, jnp.zeros((NP - N, D), jnp.float32)], axis=0)

    p1, degp = _hop_with_deg(xpad, ed)
    x1, dinv, g = _comb1(p1[0], p1[1], degp, xpad)
    (p2,) = _hop_no_deg(x1, ed)
    x2 = _comb2(p2[0], p2[1], dinv)

    return jnp.concatenate(
        [x, x1[:N], x2[:N], jnp.broadcast_to(g, (N, D))], axis=-1)


# speedup vs baseline: 5.1791x; 1.1262x over previous
import functools

import jax
import jax.numpy as jnp
from jax import lax
from jax.experimental import pallas as pl
from jax.experimental.pallas import tpu as pltpu
from jax.experimental.pallas import tpu_sc as plsc

N = 10000
D = 128
E = 320000
NC = 2
NS = 16
NW = NC * NS
C = 64
NBUF = 4
NEBUF = 8
EPW = 10240
NCHUNK = EPW // C
E2 = NW * EPW
NP = 10240
ZROW = NP // NS
L = 16


def _hop_body(with_deg, table, ed, *rest):
    if with_deg:
        out, degout = rest[0], rest[1]
        rest = rest[2:]
    else:
        out = rest[0]
        rest = rest[1:]
    ebuf = rest[0:NEBUF]
    dbuf = rest[NEBUF:NEBUF + NBUF]
    rows = rest[NEBUF + NBUF:NEBUF + 2 * NBUF]
    rest = rest[NEBUF + 2 * NBUF:]
    if with_deg:
        degl = rest[0]
        rest = rest[1:]
    acc = rest[0]
    semE = rest[1:1 + NEBUF]
    semG = rest[1 + NEBUF:1 + NEBUF + NBUF]
    semS = rest[1 + NEBUF + NBUF:1 + NEBUF + 2 * NBUF]
    cid = lax.axis_index("c")
    sid = lax.axis_index("s")
    wid = sid * NC + cid
    zv = jnp.zeros((L,), jnp.float32)

    def zero_rows(i, c):
        for f in range(D // L):
            rows[0][i, pl.ds(f * L, L)] = zv
        return c

    lax.fori_loop(0, C, zero_rows, 0)
    for k in range(ZROW // C):
        pltpu.sync_copy(rows[0], acc.at[pl.ds(sid * ZROW + k * C, C)])
    if with_deg:
        def zero_deg(i, c):
            degl[pl.ds(i * L, L)] = zv
            return c

        lax.fori_loop(0, NP // L, zero_deg, 0)
    plsc.subcore_barrier()

    def _wait_e(q):
        pltpu.make_async_copy(ed.at[wid, 0], ebuf[q], semE[q]).wait()

    def _wait_g(b):
        pltpu.make_async_copy(table.at[pl.ds(0, C)], rows[b], semG[b]).wait()

    def _wait_s(b):
        pltpu.make_async_copy(rows[b], out.at[cid, pl.ds(0, C)],
                              semS[b]).wait()

    for m in range(NEBUF):
        pltpu.async_copy(ed.at[wid, m], ebuf[m], semE[m])
    for m in range(NBUF - 1):
        _wait_e(m)
        pltpu.async_copy(table.at[ebuf[m].at[0]], rows[m], semG[m])

    def chunk(j8, c):
        for u in range(NEBUF):
            j = j8 * NEBUF + u
            b = u % NBUF
            q = u
            bk = (u + NBUF - 1) % NBUF
            qk = (u + NBUF - 1) % NEBUF
            k = j + NBUF - 1
            _wait_g(b)

            @pl.when((j >= 1) & (k < NCHUNK))
            def _():
                _wait_s(bk)

            @pl.when(k < NCHUNK)
            def _():
                _wait_e(qk)
                pltpu.async_copy(table.at[ebuf[qk].at[0]], rows[bk],
                                 semG[bk])

            def scale(g, c2):
                wv16 = plsc.bitcast(ebuf[q][2, pl.ds(g * L, L)], jnp.float32)
                for i in range(L):
                    wv = jnp.full((L,), wv16[i], jnp.float32)
                    e = g * L + i
                    for f in range(D // L):
                        rows[b][e, pl.ds(f * L, L)] = (
                            rows[b][e, pl.ds(f * L, L)] * wv)
                dv = ebuf[q][1, pl.ds(g * L, L)]
                dbuf[b][pl.ds(g * L, L)] = dv
                if with_deg:
                    plsc.addupdate_scatter(degl, [dv], wv16)
                return c2

            lax.fori_loop(0, C // L, scale, 0)
            pltpu.async_copy(rows[b], acc.at[dbuf[b]], semS[b], add=True)

            @pl.when(j + NEBUF < NCHUNK)
            def _():
                pltpu.async_copy(ed.at[wid, j + NEBUF], ebuf[q], semE[q])
        return c

    lax.fori_loop(0, NCHUNK // NEBUF, chunk, 0)
    for b in range(NBUF):
        _wait_s(b)
    plsc.subcore_barrier()

    for k in range(ZROW // C):
        sl = pl.ds(sid * ZROW + k * C, C)
        pltpu.sync_copy(acc.at[sl], out.at[cid, sl])
    if with_deg:
        pltpu.sync_copy(degl, degout.at[wid])


def _make_hop(with_deg):
    mesh = plsc.VectorSubcoreMesh(core_axis_name="c", subcore_axis_name="s")
    out_type = [jax.ShapeDtypeStruct((NC, NP, D), jnp.float32)]
    if with_deg:
        out_type.append(jax.ShapeDtypeStruct((NW, NP), jnp.float32))
    scratch = []
    scratch += [pltpu.VMEM((3, C), jnp.int32)] * NEBUF
    scratch += [pltpu.VMEM((C,), jnp.int32)] * NBUF
    scratch += [pltpu.VMEM((C, D), jnp.float32)] * NBUF
    if with_deg:
        scratch.append(pltpu.VMEM((NP,), jnp.float32))
    scratch.append(pltpu.VMEM_SHARED((NP, D), jnp.float32))
    scratch += [pltpu.SemaphoreType.DMA] * (NEBUF + 2 * NBUF)
    return pl.kernel(
        functools.partial(_hop_body, with_deg),
        out_type=out_type,
        mesh=mesh,
        scratch_types=scratch,
        compiler_params=pltpu.CompilerParams(needs_layout_passes=False),
    )


_hop_with_deg = _make_hop(True)
_hop_no_deg = _make_hop(False)

BR = 512
_ONES = None


def _comb1_body(p0, p1, dg, xb, x1o, dio, go):
    ones = jnp.ones((NW, 1), jnp.float32)
    deg = lax.dot_general(dg[...], ones, (((0,), (0,)), ((), ())),
                          preferred_element_type=jnp.float32)
    dinv = jnp.where(deg > 0, 1.0 / deg, 0.0)
    x1o[...] = (p0[...] + p1[...]) * dinv
    dio[...] = dinv
    i = pl.program_id(0)

    @pl.when(i == 0)
    def _():
        go[...] = jnp.zeros_like(go)

    go[...] += jnp.sum(xb[...], axis=0, keepdims=True)

    @pl.when(i == pl.num_programs(0) - 1)
    def _():
        go[...] = go[...] * (1.0 / N)


_comb1 = pl.pallas_call(
    _comb1_body,
    grid=(NP // BR,),
    in_specs=[
        pl.BlockSpec((BR, D), lambda i: (i, 0)),
        pl.BlockSpec((BR, D), lambda i: (i, 0)),
        pl.BlockSpec((NW, BR), lambda i: (0, i)),
        pl.BlockSpec((BR, D), lambda i: (i, 0)),
    ],
    out_specs=[
        pl.BlockSpec((BR, D), lambda i: (i, 0)),
        pl.BlockSpec((BR, 1), lambda i: (i, 0)),
        pl.BlockSpec((1, D), lambda i: (0, 0)),
    ],
    out_shape=[
        jax.ShapeDtypeStruct((NP, D), jnp.float32),
        jax.ShapeDtypeStruct((NP, 1), jnp.float32),
        jax.ShapeDtypeStruct((1, D), jnp.float32),
    ],
)


def _comb2_body(p0, p1, di, x2o):
    x2o[...] = (p0[...] + p1[...]) * di[...]


_comb2 = pl.pallas_call(
    _comb2_body,
    grid=(NP // BR,),
    in_specs=[
        pl.BlockSpec((BR, D), lambda i: (i, 0)),
        pl.BlockSpec((BR, D), lambda i: (i, 0)),
        pl.BlockSpec((BR, 1), lambda i: (i, 0)),
    ],
    out_specs=pl.BlockSpec((BR, D), lambda i: (i, 0)),
    out_shape=jax.ShapeDtypeStruct((NP, D), jnp.float32),
)


def kernel(x, edge_index, edge_weight):
    src = edge_index[0]
    dst = edge_index[1]
    pad = E2 - E
    src3 = jnp.concatenate([src, jnp.zeros((pad,), jnp.int32)]).reshape(NW, NCHUNK, C)
    dst3 = jnp.concatenate([dst, jnp.zeros((pad,), jnp.int32)]).reshape(NW, NCHUNK, C)
    ewb = jax.lax.bitcast_convert_type(
        jnp.concatenate([edge_weight, jnp.zeros((pad,), jnp.float32)]),
        jnp.int32).reshape(NW, NCHUNK, C)
    ed = jnp.stack([src3, dst3, ewb], axis=2)
    xpad = jnp.concatenate([x, jnp.zeros((NP - N, D), jnp.float32)], axis=0)

    p1, degp = _hop_with_deg(xpad, ed)
    x1, dinv, g = _comb1(p1[0], p1[1], degp, xpad)
    (p2,) = _hop_no_deg(x1, ed)
    x2 = _comb2(p2[0], p2[1], dinv)

    return jnp.concatenate(
        [x, x1[:N], x2[:N], jnp.broadcast_to(g, (N, D))], axis=-1)
